# Initial kernel scaffold; baseline (speedup 1.0000x reference)
import jax
import jax.numpy as jnp
from jax.experimental import pallas as pl


def kernel(x, node_keep_mask, source, target, edge_type, blocks):
    raise NotImplementedError("write your pallas kernel here")


# speedup vs baseline: 19.4674x; 19.4674x over previous
import functools

import jax
import jax.numpy as jnp
from jax import lax
from jax.experimental import pallas as pl
from jax.experimental.pallas import tpu as pltpu
from jax.experimental.pallas import tpu_sc as plsc

N_NODES = 10000
DIM = 128
N_REL = 4
N_BLK = 4
BS = 32
N_EDGES = 160000

NW = 32
CHUNK = 128
E2 = 2 * N_EDGES
K = -(-E2 // (NW * CHUNK))
E2P = NW * K * CHUNK
PAD = E2P - E2
ACC_ROWS = 10240
ROWS_PER_TILE = ACC_ROWS // 16


def _transform_body(x_ref, m_ref, blk_ref, y_ref, s_ref):
    xb = x_ref[...]
    for r in range(N_REL + 1):
        parts = []
        for b in range(N_BLK):
            parts.append(
                jnp.dot(xb[:, b * BS:(b + 1) * BS], blk_ref[r, b],
                        preferred_element_type=jnp.float32))
        yr = jnp.concatenate(parts, axis=1)
        if r < N_REL:
            y_ref[r] = yr
        else:
            s_ref[...] = yr * m_ref[...]


def _transform(x, maskf, blocks):
    rows = x.shape[0]
    br = 2000
    grid = rows // br
    return pl.pallas_call(
        _transform_body,
        grid=(grid,),
        in_specs=[
            pl.BlockSpec((br, DIM), lambda i: (i, 0)),
            pl.BlockSpec((br, 1), lambda i: (i, 0)),
            pl.BlockSpec((N_REL + 1, N_BLK, BS, BS), lambda i: (0, 0, 0, 0)),
        ],
        out_specs=[
            pl.BlockSpec((N_REL, br, DIM), lambda i: (0, i, 0)),
            pl.BlockSpec((br, DIM), lambda i: (i, 0)),
        ],
        out_shape=[
            jax.ShapeDtypeStruct((N_REL, rows, DIM), jnp.float32),
            jax.ShapeDtypeStruct((rows, DIM), jnp.float32),
        ],
    )(x, maskf, blocks)


def _sc_scatter(yflat, gidx, sidx, zeros):
    mesh = plsc.VectorSubcoreMesh(core_axis_name="c", subcore_axis_name="s")

    @functools.partial(
        pl.kernel,
        mesh=mesh,
        out_type=jax.ShapeDtypeStruct((2, ACC_ROWS, DIM), jnp.float32),
        scratch_types=[
            pltpu.VMEM((K, CHUNK), jnp.int32),
            pltpu.VMEM((K, CHUNK), jnp.int32),
            pltpu.VMEM((CHUNK, DIM), jnp.float32),
            pltpu.VMEM_SHARED((ACC_ROWS, DIM), jnp.float32),
            pltpu.SemaphoreType.DMA,
        ],
    )
    def body(yflat_hbm, gidx_hbm, sidx_hbm, zeros_hbm, out_hbm,
             gidx_v, sidx_v, rows_v, acc, sem):
        cid = lax.axis_index("c")
        sid = lax.axis_index("s")
        wid = cid * 16 + sid
        pltpu.sync_copy(gidx_hbm.at[wid], gidx_v)
        pltpu.sync_copy(sidx_hbm.at[wid], sidx_v)
        pltpu.sync_copy(
            zeros_hbm.at[pl.ds(sid * ROWS_PER_TILE, ROWS_PER_TILE)],
            acc.at[pl.ds(sid * ROWS_PER_TILE, ROWS_PER_TILE)])
        plsc.subcore_barrier()

        def chunk(j, carry):
            pltpu.async_copy(yflat_hbm.at[gidx_v.at[j]], rows_v, sem).wait()
            pltpu.sync_copy(rows_v, acc.at[sidx_v.at[j]], add=True)
            return carry

        lax.fori_loop(0, K, chunk, 0)
        plsc.subcore_barrier()
        pltpu.sync_copy(
            acc.at[pl.ds(sid * ROWS_PER_TILE, ROWS_PER_TILE)],
            out_hbm.at[cid].at[pl.ds(sid * ROWS_PER_TILE, ROWS_PER_TILE)])

    return body(yflat, gidx, sidx, zeros)


def _combine_body(s_ref, p0_ref, p1_ref, o_ref):
    o_ref[...] = s_ref[...] + p0_ref[...] + p1_ref[...]


def _combine(s, p0, p1):
    rows = s.shape[0]
    br = 2000
    spec = pl.BlockSpec((br, DIM), lambda i: (i, 0))
    return pl.pallas_call(
        _combine_body,
        grid=(rows // br,),
        in_specs=[spec, spec, spec],
        out_specs=spec,
        out_shape=jax.ShapeDtypeStruct((rows, DIM), jnp.float32),
    )(s, p0, p1)


def kernel(x, node_keep_mask, source, target, edge_type, blocks):
    maskf = node_keep_mask.astype(jnp.float32).reshape(N_NODES, 1)
    source = source.astype(jnp.int32)
    target = target.astype(jnp.int32)
    edge_type = edge_type.astype(jnp.int32)

    gidx = jnp.concatenate([edge_type * N_NODES + source,
                            edge_type * N_NODES + target])
    sidx = jnp.concatenate([target, source])
    pad = jnp.arange(PAD, dtype=jnp.int32)
    gidx = jnp.concatenate([gidx, (pad * 131) % (N_REL * N_NODES)])
    sidx = jnp.concatenate([sidx, N_NODES + (pad % (ACC_ROWS - N_NODES))])
    gidx = gidx.reshape(NW, K, CHUNK)
    sidx = sidx.reshape(NW, K, CHUNK)

    y, s = _transform(x, maskf, blocks)
    yflat = y.reshape(N_REL * N_NODES, DIM)
    zeros = jnp.zeros((ACC_ROWS, DIM), jnp.float32)
    p = _sc_scatter(yflat, gidx, sidx, zeros)
    return _combine(s, p[0], p[1])


# speedup vs baseline: 23.7611x; 1.2206x over previous
import functools

import jax
import jax.numpy as jnp
from jax import lax
from jax.experimental import pallas as pl
from jax.experimental.pallas import tpu as pltpu
from jax.experimental.pallas import tpu_sc as plsc

N_NODES = 10000
DIM = 128
N_REL = 4
N_BLK = 4
BS = 32
N_EDGES = 160000

NW = 32
CHUNK = 128
E2 = 2 * N_EDGES
K = 4 * (-(-E2 // (NW * CHUNK * 4)))
E2P = NW * K * CHUNK
PAD = E2P - E2
ACC_ROWS = 10112
ROWS_PER_TILE = ACC_ROWS // 16


def _transform_body(x_ref, m_ref, blk_ref, y_ref, s_ref):
    xb = x_ref[...]
    for r in range(N_REL + 1):
        parts = []
        for b in range(N_BLK):
            parts.append(
                jnp.dot(xb[:, b * BS:(b + 1) * BS], blk_ref[r, b],
                        preferred_element_type=jnp.float32))
        yr = jnp.concatenate(parts, axis=1)
        if r < N_REL:
            y_ref[r] = yr
        else:
            s_ref[...] = yr * m_ref[...]


def _transform(x, maskf, blocks):
    rows = x.shape[0]
    br = 2000
    grid = rows // br
    return pl.pallas_call(
        _transform_body,
        grid=(grid,),
        in_specs=[
            pl.BlockSpec((br, DIM), lambda i: (i, 0)),
            pl.BlockSpec((br, 1), lambda i: (i, 0)),
            pl.BlockSpec((N_REL + 1, N_BLK, BS, BS), lambda i: (0, 0, 0, 0)),
        ],
        out_specs=[
            pl.BlockSpec((N_REL, br, DIM), lambda i: (0, i, 0)),
            pl.BlockSpec((br, DIM), lambda i: (i, 0)),
        ],
        out_shape=[
            jax.ShapeDtypeStruct((N_REL, rows, DIM), jnp.float32),
            jax.ShapeDtypeStruct((rows, DIM), jnp.float32),
        ],
    )(x, maskf, blocks)


def _sc_scatter(yflat, idx, zeros):
    mesh = plsc.VectorSubcoreMesh(core_axis_name="c", subcore_axis_name="s")

    @functools.partial(
        pl.kernel,
        mesh=mesh,
        out_type=jax.ShapeDtypeStruct((2, ACC_ROWS, DIM), jnp.float32),
        scratch_types=[
            pltpu.VMEM((4, 2, CHUNK), jnp.int32),
            pltpu.VMEM((CHUNK, DIM), jnp.float32),
            pltpu.VMEM((CHUNK, DIM), jnp.float32),
            pltpu.VMEM_SHARED((ACC_ROWS, DIM), jnp.float32),
            pltpu.SemaphoreType.DMA,
            pltpu.SemaphoreType.DMA,
            pltpu.SemaphoreType.DMA,
            pltpu.SemaphoreType.DMA,
            pltpu.SemaphoreType.DMA,
            pltpu.SemaphoreType.DMA,
        ],
    )
    def body(yflat_hbm, idx_hbm, zeros_hbm, out_hbm,
             ring, rows_a, rows_b, acc, g0, g1, i0, i1, i2, i3):
        cid = lax.axis_index("c")
        sid = lax.axis_index("s")
        wid = cid * 16 + sid
        iw = idx_hbm.at[wid]
        rows = (rows_a, rows_b)
        gsem = (g0, g1)
        isem = (i0, i1, i2, i3)

        pltpu.sync_copy(
            zeros_hbm.at[pl.ds(sid * ROWS_PER_TILE, ROWS_PER_TILE)],
            acc.at[pl.ds(sid * ROWS_PER_TILE, ROWS_PER_TILE)])
        plsc.subcore_barrier()

        for u in range(4):
            pltpu.async_copy(iw.at[u], ring.at[u], isem[u])
        pltpu.make_async_copy(iw.at[0], ring.at[0], isem[0]).wait()
        pltpu.async_copy(yflat_hbm.at[ring.at[0, 0]], rows[0], gsem[0])

        def quad(i, carry):
            for u in range(4):
                j = 4 * i + u
                sl, sl1 = u, (u + 1) % 4
                bj, bj1 = rows[u % 2], rows[(u + 1) % 2]
                sj, sj1 = gsem[u % 2], gsem[(u + 1) % 2]
                pltpu.make_async_copy(yflat_hbm.at[ring.at[sl, 0]],
                                      bj, sj).wait()

                @pl.when(j + 1 < K)
                def _():
                    pltpu.make_async_copy(iw.at[j + 1], ring.at[sl1],
                                          isem[sl1]).wait()
                    pltpu.async_copy(yflat_hbm.at[ring.at[sl1, 0]],
                                     bj1, sj1)

                pltpu.sync_copy(bj, acc.at[ring.at[sl, 1]], add=True)

                @pl.when(j + 4 < K)
                def _():
                    pltpu.async_copy(iw.at[j + 4], ring.at[sl], isem[sl])
            return carry

        lax.fori_loop(0, K // 4, quad, 0)
        plsc.subcore_barrier()
        pltpu.sync_copy(
            acc.at[pl.ds(sid * ROWS_PER_TILE, ROWS_PER_TILE)],
            out_hbm.at[cid].at[pl.ds(sid * ROWS_PER_TILE, ROWS_PER_TILE)])

    return body(yflat, idx, zeros)


def _combine_body(s_ref, p0_ref, p1_ref, o_ref):
    o_ref[...] = s_ref[...] + p0_ref[...] + p1_ref[...]


def _combine(s, p0, p1):
    rows = s.shape[0]
    br = 2000
    spec = pl.BlockSpec((br, DIM), lambda i: (i, 0))
    return pl.pallas_call(
        _combine_body,
        grid=(rows // br,),
        in_specs=[spec, spec, spec],
        out_specs=spec,
        out_shape=jax.ShapeDtypeStruct((rows, DIM), jnp.float32),
    )(s, p0, p1)


def kernel(x, node_keep_mask, source, target, edge_type, blocks):
    maskf = node_keep_mask.astype(jnp.float32).reshape(N_NODES, 1)
    source = source.astype(jnp.int32)
    target = target.astype(jnp.int32)
    edge_type = edge_type.astype(jnp.int32)

    gidx = jnp.concatenate([edge_type * N_NODES + source,
                            edge_type * N_NODES + target])
    sidx = jnp.concatenate([target, source])
    pad = jnp.arange(PAD, dtype=jnp.int32)
    gidx = jnp.concatenate([gidx, (pad * 131) % (N_REL * N_NODES)])
    sidx = jnp.concatenate([sidx, N_NODES + (pad % (ACC_ROWS - N_NODES))])
    idx = jnp.stack([gidx.reshape(NW, K, CHUNK),
                     sidx.reshape(NW, K, CHUNK)], axis=2)

    y, s = _transform(x, maskf, blocks)
    yflat = y.reshape(N_REL * N_NODES, DIM)
    zeros = jnp.zeros((ACC_ROWS, DIM), jnp.float32)
    p = _sc_scatter(yflat, idx, zeros)
    return _combine(s, p[0], p[1])


# speedup vs baseline: 24.1472x; 1.0162x over previous
import functools

import jax
import jax.numpy as jnp
from jax import lax
from jax.experimental import pallas as pl
from jax.experimental.pallas import tpu as pltpu
from jax.experimental.pallas import tpu_sc as plsc

N_NODES = 10000
DIM = 128
N_REL = 4
N_BLK = 4
BS = 32
N_EDGES = 160000

NW = 32
CHUNK = 128
E2 = 2 * N_EDGES
K = 4 * (-(-E2 // (NW * CHUNK * 4)))
E2P = NW * K * CHUNK
PAD = E2P - E2
ACC_ROWS = 10112
ROWS_PER_TILE = ACC_ROWS // 16


def _transform_body(x_ref, m_ref, blk_ref, y_ref, s_ref):
    xb = x_ref[...]
    for r in range(N_REL + 1):
        parts = []
        for b in range(N_BLK):
            parts.append(
                jnp.dot(xb[:, b * BS:(b + 1) * BS], blk_ref[r, b],
                        preferred_element_type=jnp.float32))
        yr = jnp.concatenate(parts, axis=1)
        if r < N_REL:
            y_ref[r] = yr
        else:
            s_ref[...] = yr * m_ref[...]


def _transform(x, maskf, blocks):
    br = ACC_ROWS // 8
    return pl.pallas_call(
        _transform_body,
        grid=(8,),
        in_specs=[
            pl.BlockSpec((br, DIM), lambda i: (i, 0)),
            pl.BlockSpec((br, 1), lambda i: (i, 0)),
            pl.BlockSpec((N_REL + 1, N_BLK, BS, BS), lambda i: (0, 0, 0, 0)),
        ],
        out_specs=[
            pl.BlockSpec((N_REL, br, DIM), lambda i: (0, i, 0)),
            pl.BlockSpec((br, DIM), lambda i: (i, 0)),
        ],
        out_shape=[
            jax.ShapeDtypeStruct((N_REL, N_NODES, DIM), jnp.float32),
            jax.ShapeDtypeStruct((ACC_ROWS, DIM), jnp.float32),
        ],
    )(x, maskf, blocks)


def _sc_scatter(yflat, idx, spad, zeros):
    mesh = plsc.VectorSubcoreMesh(core_axis_name="c", subcore_axis_name="s")

    @functools.partial(
        pl.kernel,
        mesh=mesh,
        out_type=jax.ShapeDtypeStruct((2, ACC_ROWS, DIM), jnp.float32),
        scratch_types=[
            pltpu.VMEM((4, 2, CHUNK), jnp.int32),
            pltpu.VMEM((CHUNK, DIM), jnp.float32),
            pltpu.VMEM((CHUNK, DIM), jnp.float32),
            pltpu.VMEM_SHARED((ACC_ROWS, DIM), jnp.float32),
            pltpu.SemaphoreType.DMA,
            pltpu.SemaphoreType.DMA,
            pltpu.SemaphoreType.DMA,
            pltpu.SemaphoreType.DMA,
            pltpu.SemaphoreType.DMA,
            pltpu.SemaphoreType.DMA,
        ],
    )
    def body(yflat_hbm, idx_hbm, spad_hbm, zeros_hbm, out_hbm,
             ring, rows_a, rows_b, acc, g0, g1, i0, i1, i2, i3):
        cid = lax.axis_index("c")
        sid = lax.axis_index("s")
        wid = cid * 16 + sid
        iw = idx_hbm.at[wid]
        rows = (rows_a, rows_b)
        gsem = (g0, g1)
        isem = (i0, i1, i2, i3)

        for u in range(4):
            pltpu.async_copy(iw.at[u], ring.at[u], isem[u])
        pltpu.make_async_copy(iw.at[0], ring.at[0], isem[0]).wait()
        pltpu.async_copy(yflat_hbm.at[ring.at[0, 0]], rows[0], gsem[0])

        sl_init = pl.ds(sid * ROWS_PER_TILE, ROWS_PER_TILE)

        @pl.when(cid == 0)
        def _():
            pltpu.sync_copy(spad_hbm.at[sl_init], acc.at[sl_init])

        @pl.when(cid == 1)
        def _():
            pltpu.sync_copy(zeros_hbm.at[sl_init], acc.at[sl_init])

        plsc.subcore_barrier()


        def quad(i, carry):
            for u in range(4):
                j = 4 * i + u
                sl, sl1 = u, (u + 1) % 4
                bj, bj1 = rows[u % 2], rows[(u + 1) % 2]
                sj, sj1 = gsem[u % 2], gsem[(u + 1) % 2]
                pltpu.make_async_copy(yflat_hbm.at[ring.at[sl, 0]],
                                      bj, sj).wait()

                @pl.when(j + 1 < K)
                def _():
                    pltpu.make_async_copy(iw.at[j + 1], ring.at[sl1],
                                          isem[sl1]).wait()
                    pltpu.async_copy(yflat_hbm.at[ring.at[sl1, 0]],
                                     bj1, sj1)

                pltpu.sync_copy(bj, acc.at[ring.at[sl, 1]], add=True)

                @pl.when(j + 4 < K)
                def _():
                    pltpu.async_copy(iw.at[j + 4], ring.at[sl], isem[sl])
            return carry

        lax.fori_loop(0, K // 4, quad, 0)
        plsc.subcore_barrier()
        pltpu.sync_copy(
            acc.at[pl.ds(sid * ROWS_PER_TILE, ROWS_PER_TILE)],
            out_hbm.at[cid].at[pl.ds(sid * ROWS_PER_TILE, ROWS_PER_TILE)])

    return body(yflat, idx, spad, zeros)


def _combine_body(p0_ref, p1_ref, o_ref):
    o_ref[...] = p0_ref[...] + p1_ref[...]


def _combine(p0, p1):
    br = 2000
    spec = pl.BlockSpec((br, DIM), lambda i: (i, 0))
    return pl.pallas_call(
        _combine_body,
        grid=(N_NODES // br,),
        in_specs=[spec, spec],
        out_specs=spec,
        out_shape=jax.ShapeDtypeStruct((N_NODES, DIM), jnp.float32),
    )(p0, p1)


def kernel(x, node_keep_mask, source, target, edge_type, blocks):
    maskf = node_keep_mask.astype(jnp.float32).reshape(N_NODES, 1)
    source = source.astype(jnp.int32)
    target = target.astype(jnp.int32)
    edge_type = edge_type.astype(jnp.int32)

    gidx = jnp.concatenate([edge_type * N_NODES + source,
                            edge_type * N_NODES + target])
    sidx = jnp.concatenate([target, source])
    pad = jnp.arange(PAD, dtype=jnp.int32)
    gidx = jnp.concatenate([gidx, (pad * 131) % (N_REL * N_NODES)])
    sidx = jnp.concatenate([sidx, N_NODES + (pad % (ACC_ROWS - N_NODES))])
    idx = jnp.stack([gidx.reshape(NW, K, CHUNK),
                     sidx.reshape(NW, K, CHUNK)], axis=2)

    y, spad = _transform(x, maskf, blocks)
    yflat = y.reshape(N_REL * N_NODES, DIM)
    zeros = jnp.zeros((ACC_ROWS, DIM), jnp.float32)
    p = _sc_scatter(yflat, idx, spad, zeros)
    return _combine(p[0], p[1])
